# baseline (speedup 1.0000x reference)
import functools

import jax
import jax.numpy as jnp
from jax import lax
from jax.experimental import pallas as pl
from jax.experimental.pallas import tpu as pltpu
from jax.experimental.pallas import tpu_sc as plsc

D = 128
NC = 2
NS = 16
NW = NC * NS
CHUNK = 128
NBUF = 4
K = 3


def _gather_impl(x3d, table):
    nchunk = x3d.shape[1]
    per_w = nchunk * CHUNK
    total = NW * per_w
    ngroup = nchunk // NBUF
    mesh = plsc.VectorSubcoreMesh(core_axis_name="c", subcore_axis_name="s")

    @functools.partial(
        pl.kernel,
        mesh=mesh,
        out_type=jax.ShapeDtypeStruct((total, D), jnp.float32),
        scratch_types=[
            pltpu.VMEM((nchunk, CHUNK), jnp.int32),
            pltpu.VMEM((NBUF, CHUNK, D), jnp.float32),
            pltpu.SemaphoreType.DMA((NBUF,)),
            pltpu.SemaphoreType.DMA((NBUF,)),
        ],
    )
    def k(x_hbm, table_hbm, out_hbm, idx_v, rows_v, gsem, osem):
        wid = lax.axis_index("s") * NC + lax.axis_index("c")
        base = wid * per_w
        pltpu.sync_copy(x_hbm.at[wid], idx_v)

        def fire_gather(s, b):
            pltpu.async_copy(
                table_hbm.at[idx_v.at[s]], rows_v.at[b], gsem.at[b]
            )

        def drain_gather(s, b):
            pltpu.make_async_copy(
                table_hbm.at[idx_v.at[s]], rows_v.at[b], gsem.at[b]
            ).wait()

        def fire_store(s, b):
            pltpu.async_copy(
                rows_v.at[b], out_hbm.at[pl.ds(base + s * CHUNK, CHUNK)],
                osem.at[b],
            )

        def drain_store(b):
            pltpu.make_async_copy(
                rows_v.at[b], out_hbm.at[pl.ds(0, CHUNK)], osem.at[b]
            ).wait()

        def group(g, carry):
            for b in range(NBUF):
                s = g * NBUF + b

                @pl.when(s >= NBUF)
                def _reuse(b=b):
                    drain_store(b)

                fire_gather(s, b)
                tb = (b - K) % NBUF

                @pl.when(s >= K)
                def _retire(s=s, tb=tb):
                    drain_gather(s - K, tb)
                    fire_store(s - K, tb)

            return carry

        lax.fori_loop(0, ngroup, group, 0)
        for t in range(nchunk - K, nchunk):
            tb = t % NBUF
            drain_gather(t, tb)
            fire_store(t, tb)
        for b in range(NBUF):
            drain_store(b)

    return k(x3d, table)


def kernel(x, table):
    b, s = x.shape
    total = b * s
    x_flat = x.reshape(total)
    tc_n = 81920
    sc_n = total - tc_n
    out_sc = _gather_impl(
        x_flat[tc_n:].reshape(NW, sc_n // (NW * CHUNK), CHUNK), table
    )
    out_tc = jnp.take(table, x_flat[:tc_n], axis=0)
    return jnp.concatenate([out_tc, out_sc], axis=0).reshape(b, s, D)


# speedup vs baseline: 1.8982x; 1.8982x over previous
import functools

import jax
import jax.numpy as jnp
from jax import lax
from jax.experimental import pallas as pl
from jax.experimental.pallas import tpu as pltpu
from jax.experimental.pallas import tpu_sc as plsc

D = 128
NC = 2
NS = 16
NW = NC * NS
CHUNK = 128
NBUF = 4
K = 3


def _gather_impl(x3d, table):
    nchunk = x3d.shape[1]
    per_w = nchunk * CHUNK
    total = NW * per_w
    ngroup = nchunk // NBUF
    mesh = plsc.VectorSubcoreMesh(core_axis_name="c", subcore_axis_name="s")

    @functools.partial(
        pl.kernel,
        mesh=mesh,
        out_type=jax.ShapeDtypeStruct((total, D), jnp.float32),
        scratch_types=[
            pltpu.VMEM((nchunk, CHUNK), jnp.int32),
            pltpu.VMEM((NBUF, CHUNK, D), jnp.float32),
            pltpu.SemaphoreType.DMA((NBUF,)),
            pltpu.SemaphoreType.DMA((NBUF,)),
        ],
    )
    def k(x_hbm, table_hbm, out_hbm, idx_v, rows_v, gsem, osem):
        wid = lax.axis_index("s") * NC + lax.axis_index("c")
        base = wid * per_w
        pltpu.sync_copy(x_hbm.at[wid], idx_v)

        def fire_gather(s, b):
            pltpu.async_copy(
                table_hbm.at[idx_v.at[s]], rows_v.at[b], gsem.at[b]
            )

        def drain_gather(s, b):
            pltpu.make_async_copy(
                table_hbm.at[idx_v.at[s]], rows_v.at[b], gsem.at[b]
            ).wait()

        def fire_store(s, b):
            pltpu.async_copy(
                rows_v.at[b], out_hbm.at[pl.ds(base + s * CHUNK, CHUNK)],
                osem.at[b],
            )

        def drain_store(b):
            pltpu.make_async_copy(
                rows_v.at[b], out_hbm.at[pl.ds(0, CHUNK)], osem.at[b]
            ).wait()

        def group(g, carry):
            for b in range(NBUF):
                s = g * NBUF + b

                @pl.when(s >= NBUF)
                def _reuse(b=b):
                    drain_store(b)

                fire_gather(s, b)
                tb = (b - K) % NBUF

                @pl.when(s >= K)
                def _retire(s=s, tb=tb):
                    drain_gather(s - K, tb)
                    fire_store(s - K, tb)

            return carry

        lax.fori_loop(0, ngroup, group, 0)
        for t in range(nchunk - K, nchunk):
            tb = t % NBUF
            drain_gather(t, tb)
            fire_store(t, tb)
        for b in range(NBUF):
            drain_store(b)

    return k(x3d, table)


def kernel(x, table):
    b, s = x.shape
    total = b * s
    out = _gather_impl(x.reshape(NW, total // (NW * CHUNK), CHUNK), table)
    return out.reshape(b, s, D)


# speedup vs baseline: 1.9030x; 1.0025x over previous
import functools

import jax
import jax.numpy as jnp
from jax import lax
from jax.experimental import pallas as pl
from jax.experimental.pallas import tpu as pltpu
from jax.experimental.pallas import tpu_sc as plsc

D = 128
NC = 2
NS = 16
NW = NC * NS
CHUNK = 128
NBUF = 5
K = 3


def _gather_impl(x3d, table):
    nchunk = x3d.shape[1]
    per_w = nchunk * CHUNK
    total = NW * per_w
    ngroup = nchunk // NBUF
    mesh = plsc.VectorSubcoreMesh(core_axis_name="c", subcore_axis_name="s")

    @functools.partial(
        pl.kernel,
        mesh=mesh,
        out_type=jax.ShapeDtypeStruct((total, D), jnp.float32),
        scratch_types=[
            pltpu.VMEM((nchunk, CHUNK), jnp.int32),
            pltpu.VMEM((NBUF, CHUNK, D), jnp.float32),
            pltpu.SemaphoreType.DMA((NBUF,)),
            pltpu.SemaphoreType.DMA((NBUF,)),
        ],
    )
    def k(x_hbm, table_hbm, out_hbm, idx_v, rows_v, gsem, osem):
        wid = lax.axis_index("s") * NC + lax.axis_index("c")
        base = wid * per_w
        pltpu.sync_copy(x_hbm.at[wid], idx_v)

        def fire_gather(s, b):
            pltpu.async_copy(
                table_hbm.at[idx_v.at[s]], rows_v.at[b], gsem.at[b]
            )

        def drain_gather(s, b):
            pltpu.make_async_copy(
                table_hbm.at[idx_v.at[s]], rows_v.at[b], gsem.at[b]
            ).wait()

        def fire_store(s, b):
            pltpu.async_copy(
                rows_v.at[b], out_hbm.at[pl.ds(base + s * CHUNK, CHUNK)],
                osem.at[b],
            )

        def drain_store(b):
            pltpu.make_async_copy(
                rows_v.at[b], out_hbm.at[pl.ds(0, CHUNK)], osem.at[b]
            ).wait()

        def group(g, carry):
            for b in range(NBUF):
                s = g * NBUF + b

                @pl.when(s >= NBUF)
                def _reuse(b=b):
                    drain_store(b)

                fire_gather(s, b)
                tb = (b - K) % NBUF

                @pl.when(s >= K)
                def _retire(s=s, tb=tb):
                    drain_gather(s - K, tb)
                    fire_store(s - K, tb)

            return carry

        lax.fori_loop(0, ngroup, group, 0)
        for t in range(nchunk - K, nchunk):
            tb = t % NBUF
            drain_gather(t, tb)
            fire_store(t, tb)
        for b in range(NBUF):
            drain_store(b)

    return k(x3d, table)


def kernel(x, table):
    b, s = x.shape
    total = b * s
    out = _gather_impl(x.reshape(NW, total // (NW * CHUNK), CHUNK), table)
    return out.reshape(b, s, D)
